# baseline (speedup 1.0000x reference)
import functools
import jax
import jax.numpy as jnp
from jax import lax
from jax.experimental import pallas as pl
from jax.experimental.pallas import tpu as pltpu
from jax.experimental.pallas import tpu_sc as plsc

_B = 16384
_A = 51
_NC = 2
_NS = 16
_NW = _NC * _NS
_RPW = _B // _NW
_WPW = _RPW * _A
_CROWS = 64
_NCHUNK = _RPW // _CROWS
_CW = _CROWS * _A
_GPC = _CROWS // 16

_LN2 = 0.6931471805599453


def _log16(s):
    bits = plsc.bitcast(s, jnp.int32)
    ex = (bits >> 23) - 127
    mant = plsc.bitcast((bits & jnp.int32(0x007FFFFF)) | jnp.int32(0x3F800000),
                        jnp.float32)
    z = (mant - 1.0) / (mant + 1.0)
    z2 = z * z
    logm = 2.0 * z * (1.0 + z2 * (1.0 / 3.0 + z2 * (0.2 + z2 * (1.0 / 7.0))))
    return ex.astype(jnp.float32) * _LN2 + logm


def _worker_body(x_hbm, t_hbm, out_hbm,
                 xb0, tb0, xb1, tb1, part_sh, red,
                 sx0, st0, sx1, st1):
    c = lax.axis_index("c")
    s = lax.axis_index("s")
    wid = c * _NS + s
    rbase = wid * _RPW

    lanes = lax.iota(jnp.int32, 16)
    row_off = lanes * _A

    bufs = ((xb0, tb0, sx0, st0), (xb1, tb1, sx1, st1))

    def fire(chunk):
        xb, tb, sx, st = bufs[chunk % 2]
        r0 = rbase + chunk * _CROWS
        cx = pltpu.async_copy(x_hbm.at[pl.ds(r0, _CROWS), :], xb, sx)
        ct = pltpu.async_copy(t_hbm.at[pl.ds(r0, _CROWS), :], tb, st)
        return cx, ct

    def make_group_body(xb, tb):
        def group_body(g, tot):
            rows = g * 16 + lanes
            S = jnp.zeros((16,), jnp.float32)
            at = jnp.zeros((16,), jnp.float32)
            atx = jnp.zeros((16,), jnp.float32)
            for a in range(_A):
                cols = jnp.full((16,), a, jnp.int32)
                xv = plsc.load_gather(xb, [rows, cols])
                tv = plsc.load_gather(tb, [rows, cols])
                S = S + jnp.exp(xv)
                at = at + tv
                atx = atx + tv * xv
            return tot + at * _log16(S) - atx
        return group_body

    total = jnp.zeros((16,), jnp.float32)
    pend = fire(0)
    for chunk in range(_NCHUNK):
        cx, ct = pend
        cx.wait()
        ct.wait()
        xb, tb, _, _ = bufs[chunk % 2]
        if chunk + 1 < _NCHUNK:
            pend = fire(chunk + 1)
        total = lax.fori_loop(0, _GPC, make_group_body(xb, tb), total)

    red[...] = total
    pltpu.sync_copy(red, out_hbm.at[wid, pl.ds(0, 16)])


def _sc_loss(xflat, tflat):
    mesh = plsc.VectorSubcoreMesh(core_axis_name="c", subcore_axis_name="s")
    f = pl.kernel(
        _worker_body,
        out_type=jax.ShapeDtypeStruct((_NW, 128), jnp.float32),
        mesh=mesh,
        scratch_types=[
            pltpu.VMEM((_CROWS, _A), jnp.float32),
            pltpu.VMEM((_CROWS, _A), jnp.float32),
            pltpu.VMEM((_CROWS, _A), jnp.float32),
            pltpu.VMEM((_CROWS, _A), jnp.float32),
            pltpu.VMEM_SHARED((_NS, 16), jnp.float32),
            pltpu.VMEM((16,), jnp.float32),
            pltpu.SemaphoreType.DMA,
            pltpu.SemaphoreType.DMA,
            pltpu.SemaphoreType.DMA,
            pltpu.SemaphoreType.DMA,
        ],
        compiler_params=pltpu.CompilerParams(needs_layout_passes=False),
    )
    return f(xflat, tflat)


def kernel(current_logits, target_distribution):
    out = _sc_loss(current_logits, target_distribution)
    return jnp.sum(out[:, :16]) / _B


# speedup vs baseline: 7.7663x; 7.7663x over previous
import jax
import jax.numpy as jnp
from jax.experimental import pallas as pl
from jax.experimental.pallas import tpu as pltpu

_B = 16384
_A = 51
_NB = 2048
_NSTEP = _B // _NB


def _ce_body(x_ref, t_ref, out_ref):
    j = pl.program_id(0)

    @pl.when(j == 0)
    def _():
        out_ref[0, 0] = 0.0

    x = x_ref[...]
    t = t_ref[...]
    m = jnp.max(x, axis=0, keepdims=True)
    s = jnp.sum(jnp.exp(x - m), axis=0, keepdims=True)
    at = jnp.sum(t, axis=0, keepdims=True)
    atx = jnp.sum(t * x, axis=0, keepdims=True)
    lse = m + jnp.log(s)
    out_ref[0, 0] += jnp.sum(at * lse - atx) * (1.0 / _B)


def kernel(current_logits, target_distribution):
    xt = current_logits.T
    tt = target_distribution.T
    out = pl.pallas_call(
        _ce_body,
        grid=(_NSTEP,),
        in_specs=[
            pl.BlockSpec((_A, _NB), lambda j: (0, j)),
            pl.BlockSpec((_A, _NB), lambda j: (0, j)),
        ],
        out_specs=pl.BlockSpec(memory_space=pltpu.SMEM),
        out_shape=jax.ShapeDtypeStruct((1, 1), jnp.float32),
    )(xt, tt)
    return out[0, 0]


# speedup vs baseline: 10.0313x; 1.2916x over previous
import jax
import jax.numpy as jnp
from jax.experimental import pallas as pl
from jax.experimental.pallas import tpu as pltpu

_B = 16384
_A = 51
_NB = 4096
_NSTEP = _B // _NB


def _ce_body(x_ref, t_ref, out_ref):
    j = pl.program_id(0)

    @pl.when(j == 0)
    def _():
        out_ref[0, 0] = 0.0

    x = x_ref[...]
    t = t_ref[...]
    m = jnp.max(x, axis=0, keepdims=True)
    s = jnp.sum(jnp.exp(x - m), axis=0, keepdims=True)
    at = jnp.sum(t, axis=0, keepdims=True)
    atx = jnp.sum(t * x, axis=0, keepdims=True)
    lse = m + jnp.log(s)
    out_ref[0, 0] += jnp.sum(at * lse - atx) * (1.0 / _B)


def kernel(current_logits, target_distribution):
    xt = current_logits.T
    tt = target_distribution.T
    out = pl.pallas_call(
        _ce_body,
        grid=(_NSTEP,),
        in_specs=[
            pl.BlockSpec((_A, _NB), lambda j: (0, j)),
            pl.BlockSpec((_A, _NB), lambda j: (0, j)),
        ],
        out_specs=pl.BlockSpec(memory_space=pltpu.SMEM),
        out_shape=jax.ShapeDtypeStruct((1, 1), jnp.float32),
    )(xt, tt)
    return out[0, 0]
